# baseline (speedup 1.0000x reference)
import functools

import jax
import jax.numpy as jnp
from jax import lax
from jax.experimental import pallas as pl
from jax.experimental.pallas import tpu as pltpu
from jax.experimental.pallas import tpu_sc as plsc

B = 16384
D = 64
CH = 128

_info = plsc.get_sparse_core_info()
_NC, _NS = _info.num_cores, _info.num_subcores
NW = _NC * _NS
BPW = B // NW
NCH = BPW // CH
NB = 4
LAG = 2


def _sc_gather_body(uidx, midx, ugt, mgt, umt, mmt,
                    ug_o, mg_o, um_o, mm_o,
                    uidx_v, midx_v, b0, b1, b2, b3,
                    gs0, gs1, gs2, gs3, ws0, ws1, ws2, ws3):
    wid = lax.axis_index("s") * _NC + lax.axis_index("c")
    base = wid * BPW
    pltpu.sync_copy(uidx.at[pl.ds(wid * NCH, NCH)], uidx_v)
    pltpu.sync_copy(midx.at[pl.ds(wid * NCH, NCH)], midx_v)

    bufs = (b0, b1, b2, b3)
    gsems = (gs0, gs1, gs2, gs3)
    wsems = (ws0, ws1, ws2, ws3)
    jobs = []
    for tbl, iv, out in ((ugt, uidx_v, ug_o), (mgt, midx_v, mg_o),
                         (umt, uidx_v, um_o), (mmt, midx_v, mm_o)):
        for c in range(NCH):
            jobs.append((tbl, iv, out, c))
    nu = len(jobs)
    gh = [None] * NB
    wh = [None] * NB
    for k in range(nu + LAG):
        if k < nu:
            tbl, iv, out, c = jobs[k]
            bk = k % NB
            if k >= NB:
                wh[bk].wait()
            gh[bk] = pltpu.async_copy(tbl.at[iv.at[c]], bufs[bk], gsems[bk])
        j = k - LAG
        if 0 <= j < nu:
            tbl, iv, out, c = jobs[j]
            bj = j % NB
            gh[bj].wait()
            wh[bj] = pltpu.async_copy(
                bufs[bj], out.at[pl.ds(base + c * CH, CH)], wsems[bj])
    for b in range(NB):
        wh[b].wait()


@functools.partial(jax.jit, static_argnames=())
def _sc_gather(uidx2d, midx2d, ugt, mgt, umt, mmt):
    mesh = plsc.VectorSubcoreMesh(core_axis_name="c", subcore_axis_name="s")
    f = functools.partial(
        pl.kernel,
        mesh=mesh,
        compiler_params=pltpu.CompilerParams(use_tc_tiling_on_sc=False),
        out_type=[jax.ShapeDtypeStruct((B, D), jnp.float32)] * 4,
        scratch_types=[
            pltpu.VMEM((NCH, CH), jnp.int32),
            pltpu.VMEM((NCH, CH), jnp.int32),
        ] + [pltpu.VMEM((CH, D), jnp.float32)] * NB
          + [pltpu.SemaphoreType.DMA] * (2 * NB),
    )(_sc_gather_body)
    return f(uidx2d, midx2d, ugt, mgt, umt, mmt)


def _tc_dense_body(ug_ref, mg_ref, um_ref, mm_ref, w1u_ref, w1m_ref, b1_ref,
                   w2_ref, b2_ref, wfg_ref, wfm_ref, bf_ref, o_ref):
    um = um_ref[...]
    mm = mm_ref[...]
    h = jnp.maximum(
        jnp.dot(um, w1u_ref[...], preferred_element_type=jnp.float32)
        + jnp.dot(mm, w1m_ref[...], preferred_element_type=jnp.float32)
        + b1_ref[...][None, :], 0.0)
    m = jnp.maximum(
        jnp.dot(h, w2_ref[...], preferred_element_type=jnp.float32)
        + b2_ref[...][None, :], 0.0)
    g = ug_ref[...] * mg_ref[...]
    pred = (jnp.sum(g * wfg_ref[...][None, :], axis=-1)
            + jnp.sum(m * wfm_ref[...][None, :], axis=-1) + bf_ref[0])
    o_ref[...] = pred


def _tc_dense(ug, mg, um, mm, w1u, w1m, b1, w2t, b2, wfg, wfm, bf):
    bb = 2048
    grid = (B // bb,)
    row = lambda i: (i, 0)
    full2 = lambda i: (0, 0)
    full1 = lambda i: (0,)
    return pl.pallas_call(
        _tc_dense_body,
        grid=grid,
        in_specs=[
            pl.BlockSpec((bb, D), row),
            pl.BlockSpec((bb, D), row),
            pl.BlockSpec((bb, D), row),
            pl.BlockSpec((bb, D), row),
            pl.BlockSpec((D, D), full2),
            pl.BlockSpec((D, D), full2),
            pl.BlockSpec((D,), full1),
            pl.BlockSpec((D, D // 2), full2),
            pl.BlockSpec((D // 2,), full1),
            pl.BlockSpec((D,), full1),
            pl.BlockSpec((D // 2,), full1),
            pl.BlockSpec((1,), full1),
        ],
        out_specs=pl.BlockSpec((bb,), lambda i: (i,)),
        out_shape=jax.ShapeDtypeStruct((B,), jnp.float32),
    )(ug, mg, um, mm, w1u, w1m, b1, w2t, b2, wfg, wfm, bf)


def kernel(user_indices, movie_indices, user_gmf_table, movie_gmf_table,
           user_mlp_table, movie_mlp_table, W1, b1, W2, b2, Wf, bf):
    u2 = user_indices.astype(jnp.int32).reshape(B // CH, CH)
    m2 = movie_indices.astype(jnp.int32).reshape(B // CH, CH)
    ug, mg, um, mm = _sc_gather(u2, m2, user_gmf_table, movie_gmf_table,
                                user_mlp_table, movie_mlp_table)
    w1u = W1[:, :D].T
    w1m = W1[:, D:].T
    w2t = W2.T
    wfg = Wf[0, :D]
    wfm = Wf[0, D:]
    return _tc_dense(ug, mg, um, mm, w1u, w1m, b1, w2t, b2, wfg, wfm, bf)
